# baseline (speedup 1.0000x reference)
import functools

import jax
import jax.numpy as jnp
from jax import lax
from jax.experimental import pallas as pl
from jax.experimental.pallas import tpu as pltpu
from jax.experimental.pallas import tpu_sc as plsc

B, N, C = 4, 32768, 64
M = B * N
H = M // 2
NC, NS = 2, 16
NW = NC * NS
RPW = M // NW
F = 128
CHUNKS = RPW // F
BN = 2048
NJ = N // BN
BR = 2048
KB = H // BR
NR = N // BR
BN2 = 4096
NJ2 = N // BN2


def _make_sc_gather_max():
    mesh = plsc.VectorSubcoreMesh(core_axis_name="c", subcore_axis_name="s")

    row_buf = pltpu.VMEM((F, C), jnp.float32)

    @functools.partial(
        pl.kernel,
        mesh=mesh,
        out_type=jax.ShapeDtypeStruct((H, 2 * C), jnp.float32),
        compiler_params=pltpu.CompilerParams(use_tc_tiling_on_sc=False,
                                             needs_layout_passes=False),
        scratch_types=[
            pltpu.VMEM((3 * RPW,), jnp.int32),
            pltpu.VMEM((RPW,), jnp.int32),
            pltpu.VMEM((RPW,), jnp.int32),
            pltpu.VMEM((RPW,), jnp.int32),
            row_buf, row_buf, row_buf, row_buf,
            row_buf, row_buf, row_buf, row_buf,
            pltpu.SemaphoreType.DMA,
            pltpu.SemaphoreType.DMA,
            pltpu.SemaphoreType.DMA,
        ],
    )
    def sc_gather_max(table_hbm, nbrflat_hbm, out_hbm,
                      raw, idx0, idx1, idx2,
                      g0a, g1a, g2a, acca,
                      g0b, g1b, g2b, accb,
                      sema, semb, wsem):
        wid = lax.axis_index("s") * NC + lax.axis_index("c")
        wbase = wid * RPW
        half = wid // (NW // 2)
        rbase = wbase - half * H
        cofs = half * C
        bufs = ((g0a, g1a, g2a, acca, sema), (g0b, g1b, g2b, accb, semb))

        pltpu.sync_copy(nbrflat_hbm.at[pl.ds(wbase * 3, 3 * RPW)], raw)
        iot = lax.iota(jnp.int32, 16)
        boff = (wid // (NW // B)) * N

        def deg_body(g16, carry):
            k = g16 * 16
            src = 3 * (k + iot)
            idx0[pl.ds(k, 16)] = plsc.load_gather(raw, [src]) + boff
            idx1[pl.ds(k, 16)] = plsc.load_gather(raw, [src + 1]) + boff
            idx2[pl.ds(k, 16)] = plsc.load_gather(raw, [src + 2]) + boff
            return carry

        lax.fori_loop(0, RPW // 16, deg_body, 0, unroll=4)

        def issue(g, s):
            g0, g1, g2, acc, sem = bufs[s]
            sl = pl.ds(g * F, F)
            pltpu.async_copy(table_hbm.at[idx0.at[sl]], g0, sem)
            pltpu.async_copy(table_hbm.at[idx1.at[sl]], g1, sem)
            pltpu.async_copy(table_hbm.at[idx2.at[sl]], g2, sem)
            pltpu.async_copy(table_hbm.at[pl.ds(wbase + g * F, F)], acc, sem)

        def drain(s):
            g0, g1, g2, acc, sem = bufs[s]
            for dst in (g0, g1, g2, acc):
                pltpu.make_async_copy(table_hbm.at[pl.ds(0, F)], dst, sem).wait()

        def compute(s):
            g0, g1, g2, acc, _ = bufs[s]

            def row_body(r, rc):
                for c in range(C // 16):
                    sl = pl.ds(c * 16, 16)
                    m01 = jnp.maximum(g0[r, sl], g1[r, sl])
                    m23 = jnp.maximum(g2[r, sl], acc[r, sl])
                    acc[r, sl] = jnp.maximum(m01, m23)
                return rc

            lax.fori_loop(0, F, row_body, 0, unroll=2)

        def writeback(g, s):
            acc = bufs[s][3]
            pltpu.async_copy(
                acc, out_hbm.at[pl.ds(rbase + g * F, F), pl.ds(cofs, C)], wsem)

        def wb_wait():
            pltpu.make_async_copy(
                acca, out_hbm.at[pl.ds(0, F), pl.ds(0, C)], wsem).wait()

        issue(0, 0)

        def pair_body(p, carry):
            for s in range(2):
                g = 2 * p + s

                @pl.when(g >= 1)
                def _():
                    wb_wait()

                @pl.when(g + 1 < CHUNKS)
                def _():
                    issue(g + 1, 1 - s)

                drain(s)
                compute(s)
                writeback(g, s)
            return carry

        lax.fori_loop(0, CHUNKS // 2, pair_body, 0)
        wb_wait()

    return sc_gather_max


_SC_CACHE = []


def _sc_gather_max(table, nbrflat):
    if not _SC_CACHE:
        _SC_CACHE.append(_make_sc_gather_max())
    return _SC_CACHE[0](table, nbrflat)


def _tc_pass1a_body(sp_ref, st_ref, w1s_ref, w1t_ref, y1_ref, stats_ref):
    b = pl.program_id(0)
    j = pl.program_id(1)
    y1 = (jnp.dot(w1s_ref[...], sp_ref[0], preferred_element_type=jnp.float32)
          + jnp.dot(w1t_ref[...], st_ref[0], preferred_element_type=jnp.float32))
    y1_ref[0] = y1.astype(jnp.bfloat16)
    blk = jnp.stack([jnp.sum(y1, axis=1), jnp.sum(y1 * y1, axis=1)])

    @pl.when((b == 0) & (j == 0))
    def _():
        stats_ref[...] = jnp.zeros_like(stats_ref)

    stats_ref[...] += blk


def _tc_pass1b_body(smax_ref, w2e_ref, w2o_ref, y2_ref, stats_ref):
    k = pl.program_id(0)
    x = smax_ref[...]
    y2a = lax.dot_general(w2e_ref[...], x,
                          dimension_numbers=(((1,), (1,)), ((), ())),
                          preferred_element_type=jnp.float32)
    y2b = lax.dot_general(w2o_ref[...], x,
                          dimension_numbers=(((1,), (1,)), ((), ())),
                          preferred_element_type=jnp.float32)
    y2_ref[0, 0] = y2a.astype(jnp.bfloat16)
    y2_ref[1, 0] = y2b.astype(jnp.bfloat16)
    blk = jnp.stack([jnp.sum(y2a, axis=1) + jnp.sum(y2b, axis=1),
                     jnp.sum(y2a * y2a, axis=1) + jnp.sum(y2b * y2b, axis=1)])

    @pl.when(k == 0)
    def _():
        stats_ref[...] = jnp.zeros_like(stats_ref)

    stats_ref[...] += blk


def _tc_pass2_body(y1_ref, y2_ref, prm_ref, sp_out_ref, st_out_ref):
    sc1 = prm_ref[0, :]
    sh1 = prm_ref[1, :]
    sc2 = prm_ref[2, :]
    sh2 = prm_ref[3, :]
    y1 = y1_ref[0].astype(jnp.float32)
    y2 = y2_ref[0, 0].astype(jnp.float32)
    sp_out_ref[0] = jnp.maximum(y1 * sc1[:, None] + sh1[:, None], 0.0)
    st_out_ref[0] = jnp.maximum(y2 * sc2[:, None] + sh2[:, None], 0.0)


def _chan_blocks(bshape):
    return pl.BlockSpec(bshape, lambda b, j: (b, 0, j))


def _full_block(shape):
    return pl.BlockSpec(shape, lambda b, j: tuple(0 for _ in shape))


_tc_pass1a = pl.pallas_call(
    _tc_pass1a_body,
    grid=(B, NJ),
    in_specs=[
        _chan_blocks((1, C, BN)),
        _chan_blocks((1, C, BN)),
        _full_block((C, C)),
        _full_block((C, C)),
    ],
    out_specs=[
        _chan_blocks((1, C, BN)),
        _full_block((2, C)),
    ],
    out_shape=[
        jax.ShapeDtypeStruct((B, C, N), jnp.bfloat16),
        jax.ShapeDtypeStruct((2, C), jnp.float32),
    ],
)

_tc_pass1b = pl.pallas_call(
    _tc_pass1b_body,
    grid=(KB,),
    in_specs=[
        pl.BlockSpec((BR, 2 * C), lambda k: (k, 0)),
        pl.BlockSpec((C, 2 * C), lambda k: (0, 0)),
        pl.BlockSpec((C, 2 * C), lambda k: (0, 0)),
    ],
    out_specs=[
        pl.BlockSpec((2, 1, C, BR), lambda k: (0, k // NR, 0, k % NR)),
        pl.BlockSpec((2, C), lambda k: (0, 0)),
    ],
    out_shape=[
        jax.ShapeDtypeStruct((2, 2, C, N), jnp.bfloat16),
        jax.ShapeDtypeStruct((2, C), jnp.float32),
    ],
)

_tc_pass2 = pl.pallas_call(
    _tc_pass2_body,
    grid=(B, NJ2),
    in_specs=[
        _chan_blocks((1, C, BN2)),
        pl.BlockSpec((1, 1, C, BN2), lambda b, j: (b // 2, b % 2, 0, j)),
        _full_block((4, C)),
    ],
    out_specs=[
        _chan_blocks((1, C, BN2)),
        _chan_blocks((1, C, BN2)),
    ],
    out_shape=[
        jax.ShapeDtypeStruct((B, C, N), jnp.float32),
        jax.ShapeDtypeStruct((B, C, N), jnp.float32),
    ],
)


def kernel(spatial_fea, structural_fea, neighbor_index,
           W_comb, b_comb, g_comb, be_comb,
           W_agg, b_agg, g_agg, be_agg):
    table = structural_fea.transpose(0, 2, 1).reshape(M, C)
    nbrflat = neighbor_index.reshape(3 * M)

    smax = _sc_gather_max(table, nbrflat)

    w1s = W_comb[:, :C]
    w1t = W_comb[:, C:]
    zeros = jnp.zeros((C, C), jnp.float32)
    w2e = jnp.concatenate([W_agg, zeros], axis=1)
    w2o = jnp.concatenate([zeros, W_agg], axis=1)
    y1, stats1 = _tc_pass1a(spatial_fea, structural_fea, w1s, w1t)
    y2, stats2 = _tc_pass1b(smax, w2e, w2o)

    inv_m = 1.0 / M
    mean1 = stats1[0] * inv_m
    var1 = stats1[1] * inv_m - mean1 * mean1
    mean2 = stats2[0] * inv_m
    var2 = stats2[1] * inv_m - mean2 * mean2
    sc1 = g_comb * lax.rsqrt(var1 + 1e-5)
    sh1 = be_comb - sc1 * mean1
    sc2 = g_agg * lax.rsqrt(var2 + 1e-5)
    sh2 = be_agg - sc2 * mean2
    prm = jnp.stack([sc1, sh1, sc2, sh2])

    sp_out, st_out = _tc_pass2(y1, y2, prm)
    return (sp_out, st_out)


# speedup vs baseline: 1.2549x; 1.2549x over previous
import functools

import jax
import jax.numpy as jnp
from jax import lax
from jax.experimental import pallas as pl
from jax.experimental.pallas import tpu as pltpu
from jax.experimental.pallas import tpu_sc as plsc

B, N, C = 4, 32768, 64
M = B * N
H = M // 2
NC, NS = 2, 16
NW = NC * NS
RPW = M // NW
F = 128
CHUNKS = RPW // F
BN = 2048
NJ = N // BN
BR = 2048
KB = H // BR
NR = N // BR
BN2 = 4096
NJ2 = N // BN2


def _make_sc_gather_max():
    mesh = plsc.VectorSubcoreMesh(core_axis_name="c", subcore_axis_name="s")

    row_buf = pltpu.VMEM((F, C), jnp.float32)

    @functools.partial(
        pl.kernel,
        mesh=mesh,
        out_type=jax.ShapeDtypeStruct((H, 2 * C), jnp.float32),
        compiler_params=pltpu.CompilerParams(use_tc_tiling_on_sc=False,
                                             needs_layout_passes=False),
        scratch_types=[
            pltpu.VMEM((RPW,), jnp.int32),
            pltpu.VMEM((RPW,), jnp.int32),
            pltpu.VMEM((RPW,), jnp.int32),
            row_buf, row_buf, row_buf, row_buf,
            row_buf, row_buf, row_buf, row_buf,
            pltpu.SemaphoreType.DMA,
            pltpu.SemaphoreType.DMA,
            pltpu.SemaphoreType.DMA,
        ],
    )
    def sc_gather_max(table_hbm, nbr0_hbm, nbr1_hbm, nbr2_hbm, out_hbm,
                      idx0, idx1, idx2,
                      g0a, g1a, g2a, acca,
                      g0b, g1b, g2b, accb,
                      sema, semb, wsem):
        wid = lax.axis_index("s") * NC + lax.axis_index("c")
        wbase = wid * RPW
        half = wid // (NW // 2)
        rbase = wbase - half * H
        cofs = half * C
        bufs = ((g0a, g1a, g2a, acca, sema), (g0b, g1b, g2b, accb, semb))

        pltpu.sync_copy(nbr0_hbm.at[pl.ds(wbase, RPW)], idx0)
        pltpu.sync_copy(nbr1_hbm.at[pl.ds(wbase, RPW)], idx1)
        pltpu.sync_copy(nbr2_hbm.at[pl.ds(wbase, RPW)], idx2)

        def issue(g, s):
            g0, g1, g2, acc, sem = bufs[s]
            sl = pl.ds(g * F, F)
            pltpu.async_copy(table_hbm.at[idx0.at[sl]], g0, sem)
            pltpu.async_copy(table_hbm.at[idx1.at[sl]], g1, sem)
            pltpu.async_copy(table_hbm.at[idx2.at[sl]], g2, sem)
            pltpu.async_copy(table_hbm.at[pl.ds(wbase + g * F, F)], acc, sem)

        def drain(s):
            g0, g1, g2, acc, sem = bufs[s]
            for dst in (g0, g1, g2, acc):
                pltpu.make_async_copy(table_hbm.at[pl.ds(0, F)], dst, sem).wait()

        def compute(s):
            g0, g1, g2, acc, _ = bufs[s]

            def row_body(r, rc):
                for c in range(C // 16):
                    sl = pl.ds(c * 16, 16)
                    m01 = jnp.maximum(g0[r, sl], g1[r, sl])
                    m23 = jnp.maximum(g2[r, sl], acc[r, sl])
                    acc[r, sl] = jnp.maximum(m01, m23)
                return rc

            lax.fori_loop(0, F, row_body, 0, unroll=2)

        def writeback(g, s):
            acc = bufs[s][3]
            pltpu.async_copy(
                acc, out_hbm.at[pl.ds(rbase + g * F, F), pl.ds(cofs, C)], wsem)

        def wb_wait():
            pltpu.make_async_copy(
                acca, out_hbm.at[pl.ds(0, F), pl.ds(0, C)], wsem).wait()

        issue(0, 0)

        def pair_body(p, carry):
            for s in range(2):
                g = 2 * p + s

                @pl.when(g >= 1)
                def _():
                    wb_wait()

                @pl.when(g + 1 < CHUNKS)
                def _():
                    issue(g + 1, 1 - s)

                drain(s)
                compute(s)
                writeback(g, s)
            return carry

        lax.fori_loop(0, CHUNKS // 2, pair_body, 0)
        wb_wait()

    return sc_gather_max


_SC_CACHE = []


def _sc_gather_max(table, nbr0, nbr1, nbr2):
    if not _SC_CACHE:
        _SC_CACHE.append(_make_sc_gather_max())
    return _SC_CACHE[0](table, nbr0, nbr1, nbr2)


def _tc_pass1a_body(sp_ref, st_ref, w1s_ref, w1t_ref, y1_ref, stats_ref):
    b = pl.program_id(0)
    j = pl.program_id(1)
    y1 = (jnp.dot(w1s_ref[...], sp_ref[0], preferred_element_type=jnp.float32)
          + jnp.dot(w1t_ref[...], st_ref[0], preferred_element_type=jnp.float32))
    y1_ref[0] = y1.astype(jnp.bfloat16)
    blk = jnp.stack([jnp.sum(y1, axis=1), jnp.sum(y1 * y1, axis=1)])

    @pl.when((b == 0) & (j == 0))
    def _():
        stats_ref[...] = jnp.zeros_like(stats_ref)

    stats_ref[...] += blk


def _tc_pass1b_body(smax_ref, w2e_ref, w2o_ref, y2_ref, stats_ref):
    k = pl.program_id(0)
    x = smax_ref[...]
    y2a = lax.dot_general(w2e_ref[...], x,
                          dimension_numbers=(((1,), (1,)), ((), ())),
                          preferred_element_type=jnp.float32)
    y2b = lax.dot_general(w2o_ref[...], x,
                          dimension_numbers=(((1,), (1,)), ((), ())),
                          preferred_element_type=jnp.float32)
    y2_ref[0, 0] = y2a.astype(jnp.bfloat16)
    y2_ref[1, 0] = y2b.astype(jnp.bfloat16)
    blk = jnp.stack([jnp.sum(y2a, axis=1) + jnp.sum(y2b, axis=1),
                     jnp.sum(y2a * y2a, axis=1) + jnp.sum(y2b * y2b, axis=1)])

    @pl.when(k == 0)
    def _():
        stats_ref[...] = jnp.zeros_like(stats_ref)

    stats_ref[...] += blk


def _tc_pass2_body(y1_ref, y2_ref, prm_ref, sp_out_ref, st_out_ref):
    sc1 = prm_ref[0, :]
    sh1 = prm_ref[1, :]
    sc2 = prm_ref[2, :]
    sh2 = prm_ref[3, :]
    y1 = y1_ref[0].astype(jnp.float32)
    y2 = y2_ref[0, 0].astype(jnp.float32)
    sp_out_ref[0] = jnp.maximum(y1 * sc1[:, None] + sh1[:, None], 0.0)
    st_out_ref[0] = jnp.maximum(y2 * sc2[:, None] + sh2[:, None], 0.0)


def _chan_blocks(bshape):
    return pl.BlockSpec(bshape, lambda b, j: (b, 0, j))


def _full_block(shape):
    return pl.BlockSpec(shape, lambda b, j: tuple(0 for _ in shape))


_tc_pass1a = pl.pallas_call(
    _tc_pass1a_body,
    grid=(B, NJ),
    in_specs=[
        _chan_blocks((1, C, BN)),
        _chan_blocks((1, C, BN)),
        _full_block((C, C)),
        _full_block((C, C)),
    ],
    out_specs=[
        _chan_blocks((1, C, BN)),
        _full_block((2, C)),
    ],
    out_shape=[
        jax.ShapeDtypeStruct((B, C, N), jnp.bfloat16),
        jax.ShapeDtypeStruct((2, C), jnp.float32),
    ],
)

_tc_pass1b = pl.pallas_call(
    _tc_pass1b_body,
    grid=(KB,),
    in_specs=[
        pl.BlockSpec((BR, 2 * C), lambda k: (k, 0)),
        pl.BlockSpec((C, 2 * C), lambda k: (0, 0)),
        pl.BlockSpec((C, 2 * C), lambda k: (0, 0)),
    ],
    out_specs=[
        pl.BlockSpec((2, 1, C, BR), lambda k: (0, k // NR, 0, k % NR)),
        pl.BlockSpec((2, C), lambda k: (0, 0)),
    ],
    out_shape=[
        jax.ShapeDtypeStruct((2, 2, C, N), jnp.bfloat16),
        jax.ShapeDtypeStruct((2, C), jnp.float32),
    ],
)

_tc_pass2 = pl.pallas_call(
    _tc_pass2_body,
    grid=(B, NJ2),
    in_specs=[
        _chan_blocks((1, C, BN2)),
        pl.BlockSpec((1, 1, C, BN2), lambda b, j: (b // 2, b % 2, 0, j)),
        _full_block((4, C)),
    ],
    out_specs=[
        _chan_blocks((1, C, BN2)),
        _chan_blocks((1, C, BN2)),
    ],
    out_shape=[
        jax.ShapeDtypeStruct((B, C, N), jnp.float32),
        jax.ShapeDtypeStruct((B, C, N), jnp.float32),
    ],
)


def kernel(spatial_fea, structural_fea, neighbor_index,
           W_comb, b_comb, g_comb, be_comb,
           W_agg, b_agg, g_agg, be_agg):
    table = structural_fea.transpose(0, 2, 1).reshape(M, C)
    offs = (jnp.arange(B, dtype=jnp.int32) * N)[:, None]
    n0 = (neighbor_index[:, :, 0] + offs).reshape(M)
    n1 = (neighbor_index[:, :, 1] + offs).reshape(M)
    n2 = (neighbor_index[:, :, 2] + offs).reshape(M)

    smax = _sc_gather_max(table, n0, n1, n2)

    w1s = W_comb[:, :C]
    w1t = W_comb[:, C:]
    zeros = jnp.zeros((C, C), jnp.float32)
    w2e = jnp.concatenate([W_agg, zeros], axis=1)
    w2o = jnp.concatenate([zeros, W_agg], axis=1)
    y1, stats1 = _tc_pass1a(spatial_fea, structural_fea, w1s, w1t)
    y2, stats2 = _tc_pass1b(smax, w2e, w2o)

    inv_m = 1.0 / M
    mean1 = stats1[0] * inv_m
    var1 = stats1[1] * inv_m - mean1 * mean1
    mean2 = stats2[0] * inv_m
    var2 = stats2[1] * inv_m - mean2 * mean2
    sc1 = g_comb * lax.rsqrt(var1 + 1e-5)
    sh1 = be_comb - sc1 * mean1
    sc2 = g_agg * lax.rsqrt(var2 + 1e-5)
    sh2 = be_agg - sc2 * mean2
    prm = jnp.stack([sc1, sh1, sc2, sh2])

    sp_out, st_out = _tc_pass2(y1, y2, prm)
    return (sp_out, st_out)


# speedup vs baseline: 1.2875x; 1.0260x over previous
import functools

import jax
import jax.numpy as jnp
from jax import lax
from jax.experimental import pallas as pl
from jax.experimental.pallas import tpu as pltpu
from jax.experimental.pallas import tpu_sc as plsc

B, N, C = 4, 32768, 64
M = B * N
H = M // 2
NC, NS = 2, 16
NW = NC * NS
RPW = M // NW
F = 128
CHUNKS = RPW // F
BN = 2048
NJ = N // BN
BR = 4096
KB = H // BR
NR = N // BR
BN2 = 4096
NJ2 = N // BN2


def _make_sc_gather_max():
    mesh = plsc.VectorSubcoreMesh(core_axis_name="c", subcore_axis_name="s")

    row_buf = pltpu.VMEM((F, C), jnp.float32)

    @functools.partial(
        pl.kernel,
        mesh=mesh,
        out_type=jax.ShapeDtypeStruct((H, 2 * C), jnp.float32),
        compiler_params=pltpu.CompilerParams(use_tc_tiling_on_sc=False,
                                             needs_layout_passes=False),
        scratch_types=[
            pltpu.VMEM((RPW,), jnp.int32),
            pltpu.VMEM((RPW,), jnp.int32),
            pltpu.VMEM((RPW,), jnp.int32),
            row_buf, row_buf, row_buf, row_buf,
            row_buf, row_buf, row_buf, row_buf,
            pltpu.SemaphoreType.DMA,
            pltpu.SemaphoreType.DMA,
            pltpu.SemaphoreType.DMA,
        ],
    )
    def sc_gather_max(table_hbm, nbr0_hbm, nbr1_hbm, nbr2_hbm, out_hbm,
                      idx0, idx1, idx2,
                      g0a, g1a, g2a, acca,
                      g0b, g1b, g2b, accb,
                      sema, semb, wsem):
        wid = lax.axis_index("s") * NC + lax.axis_index("c")
        wbase = wid * RPW
        half = wid // (NW // 2)
        rbase = wbase - half * H
        cofs = half * C
        bufs = ((g0a, g1a, g2a, acca, sema), (g0b, g1b, g2b, accb, semb))

        pltpu.sync_copy(nbr0_hbm.at[pl.ds(wbase, RPW)], idx0)
        pltpu.sync_copy(nbr1_hbm.at[pl.ds(wbase, RPW)], idx1)
        pltpu.sync_copy(nbr2_hbm.at[pl.ds(wbase, RPW)], idx2)

        def issue(g, s):
            g0, g1, g2, acc, sem = bufs[s]
            sl = pl.ds(g * F, F)
            pltpu.async_copy(table_hbm.at[idx0.at[sl]], g0, sem)
            pltpu.async_copy(table_hbm.at[idx1.at[sl]], g1, sem)
            pltpu.async_copy(table_hbm.at[idx2.at[sl]], g2, sem)
            pltpu.async_copy(table_hbm.at[pl.ds(wbase + g * F, F)], acc, sem)

        def drain(s):
            g0, g1, g2, acc, sem = bufs[s]
            for dst in (g0, g1, g2, acc):
                pltpu.make_async_copy(table_hbm.at[pl.ds(0, F)], dst, sem).wait()

        def compute(s):
            g0, g1, g2, acc, _ = bufs[s]

            def row_body(r, rc):
                for c in range(C // 16):
                    sl = pl.ds(c * 16, 16)
                    m01 = jnp.maximum(g0[r, sl], g1[r, sl])
                    m23 = jnp.maximum(g2[r, sl], acc[r, sl])
                    acc[r, sl] = jnp.maximum(m01, m23)
                return rc

            lax.fori_loop(0, F, row_body, 0, unroll=2)

        def writeback(g, s):
            acc = bufs[s][3]
            pltpu.async_copy(
                acc, out_hbm.at[pl.ds(rbase + g * F, F), pl.ds(cofs, C)], wsem)

        def wb_wait():
            pltpu.make_async_copy(
                acca, out_hbm.at[pl.ds(0, F), pl.ds(0, C)], wsem).wait()

        issue(0, 0)

        def pair_body(p, carry):
            for s in range(2):
                g = 2 * p + s

                @pl.when(g >= 1)
                def _():
                    wb_wait()

                @pl.when(g + 1 < CHUNKS)
                def _():
                    issue(g + 1, 1 - s)

                drain(s)
                compute(s)
                writeback(g, s)
            return carry

        lax.fori_loop(0, CHUNKS // 2, pair_body, 0)
        wb_wait()

    return sc_gather_max


_SC_CACHE = []


def _sc_gather_max(table, nbr0, nbr1, nbr2):
    if not _SC_CACHE:
        _SC_CACHE.append(_make_sc_gather_max())
    return _SC_CACHE[0](table, nbr0, nbr1, nbr2)


def _tc_pass1a_body(sp_ref, st_ref, w1s_ref, w1t_ref, y1_ref, stats_ref):
    b = pl.program_id(0)
    j = pl.program_id(1)
    y1 = (jnp.dot(w1s_ref[...], sp_ref[0], preferred_element_type=jnp.float32)
          + jnp.dot(w1t_ref[...], st_ref[0], preferred_element_type=jnp.float32))
    y1_ref[0] = y1.astype(jnp.bfloat16)
    blk = jnp.stack([jnp.sum(y1, axis=1), jnp.sum(y1 * y1, axis=1)])

    @pl.when((b == 0) & (j == 0))
    def _():
        stats_ref[...] = jnp.zeros_like(stats_ref)

    stats_ref[...] += blk


def _tc_pass1b_body(smax_ref, w2e_ref, w2o_ref, y2_ref, stats_ref):
    k = pl.program_id(0)
    x = smax_ref[...]
    y2a = lax.dot_general(w2e_ref[...], x,
                          dimension_numbers=(((1,), (1,)), ((), ())),
                          preferred_element_type=jnp.float32)
    y2b = lax.dot_general(w2o_ref[...], x,
                          dimension_numbers=(((1,), (1,)), ((), ())),
                          preferred_element_type=jnp.float32)
    y2_ref[0, 0] = y2a.astype(jnp.bfloat16)
    y2_ref[1, 0] = y2b.astype(jnp.bfloat16)
    blk = jnp.stack([jnp.sum(y2a, axis=1) + jnp.sum(y2b, axis=1),
                     jnp.sum(y2a * y2a, axis=1) + jnp.sum(y2b * y2b, axis=1)])

    @pl.when(k == 0)
    def _():
        stats_ref[...] = jnp.zeros_like(stats_ref)

    stats_ref[...] += blk


def _tc_pass2_body(y1_ref, y2_ref, prm_ref, sp_out_ref, st_out_ref):
    sc1 = prm_ref[0, :]
    sh1 = prm_ref[1, :]
    sc2 = prm_ref[2, :]
    sh2 = prm_ref[3, :]
    y1 = y1_ref[0].astype(jnp.float32)
    y2 = y2_ref[0, 0].astype(jnp.float32)
    sp_out_ref[0] = jnp.maximum(y1 * sc1[:, None] + sh1[:, None], 0.0)
    st_out_ref[0] = jnp.maximum(y2 * sc2[:, None] + sh2[:, None], 0.0)


def _chan_blocks(bshape):
    return pl.BlockSpec(bshape, lambda b, j: (b, 0, j))


def _full_block(shape):
    return pl.BlockSpec(shape, lambda b, j: tuple(0 for _ in shape))


_tc_pass1a = pl.pallas_call(
    _tc_pass1a_body,
    grid=(B, NJ2),
    in_specs=[
        _chan_blocks((1, C, BN2)),
        _chan_blocks((1, C, BN2)),
        _full_block((C, C)),
        _full_block((C, C)),
    ],
    out_specs=[
        _chan_blocks((1, C, BN2)),
        _full_block((2, C)),
    ],
    out_shape=[
        jax.ShapeDtypeStruct((B, C, N), jnp.bfloat16),
        jax.ShapeDtypeStruct((2, C), jnp.float32),
    ],
)

_tc_pass1b = pl.pallas_call(
    _tc_pass1b_body,
    grid=(KB,),
    in_specs=[
        pl.BlockSpec((BR, 2 * C), lambda k: (k, 0)),
        pl.BlockSpec((C, 2 * C), lambda k: (0, 0)),
        pl.BlockSpec((C, 2 * C), lambda k: (0, 0)),
    ],
    out_specs=[
        pl.BlockSpec((2, 1, C, BR), lambda k: (0, k // NR, 0, k % NR)),
        pl.BlockSpec((2, C), lambda k: (0, 0)),
    ],
    out_shape=[
        jax.ShapeDtypeStruct((2, 2, C, N), jnp.bfloat16),
        jax.ShapeDtypeStruct((2, C), jnp.float32),
    ],
)

_tc_pass2 = pl.pallas_call(
    _tc_pass2_body,
    grid=(B, NJ2),
    in_specs=[
        _chan_blocks((1, C, BN2)),
        pl.BlockSpec((1, 1, C, BN2), lambda b, j: (b // 2, b % 2, 0, j)),
        _full_block((4, C)),
    ],
    out_specs=[
        _chan_blocks((1, C, BN2)),
        _chan_blocks((1, C, BN2)),
    ],
    out_shape=[
        jax.ShapeDtypeStruct((B, C, N), jnp.float32),
        jax.ShapeDtypeStruct((B, C, N), jnp.float32),
    ],
)


def kernel(spatial_fea, structural_fea, neighbor_index,
           W_comb, b_comb, g_comb, be_comb,
           W_agg, b_agg, g_agg, be_agg):
    table = structural_fea.transpose(0, 2, 1).reshape(M, C)
    offs = (jnp.arange(B, dtype=jnp.int32) * N)[:, None]
    n0 = (neighbor_index[:, :, 0] + offs).reshape(M)
    n1 = (neighbor_index[:, :, 1] + offs).reshape(M)
    n2 = (neighbor_index[:, :, 2] + offs).reshape(M)

    smax = _sc_gather_max(table, n0, n1, n2)

    w1s = W_comb[:, :C]
    w1t = W_comb[:, C:]
    zeros = jnp.zeros((C, C), jnp.float32)
    w2e = jnp.concatenate([W_agg, zeros], axis=1)
    w2o = jnp.concatenate([zeros, W_agg], axis=1)
    y1, stats1 = _tc_pass1a(spatial_fea, structural_fea, w1s, w1t)
    y2, stats2 = _tc_pass1b(smax, w2e, w2o)

    inv_m = 1.0 / M
    mean1 = stats1[0] * inv_m
    var1 = stats1[1] * inv_m - mean1 * mean1
    mean2 = stats2[0] * inv_m
    var2 = stats2[1] * inv_m - mean2 * mean2
    sc1 = g_comb * lax.rsqrt(var1 + 1e-5)
    sh1 = be_comb - sc1 * mean1
    sc2 = g_agg * lax.rsqrt(var2 + 1e-5)
    sh2 = be_agg - sc2 * mean2
    prm = jnp.stack([sc1, sh1, sc2, sh2])

    sp_out, st_out = _tc_pass2(y1, y2, prm)
    return (sp_out, st_out)
